# initial kernel scaffold (unmeasured)
import jax
import jax.numpy as jnp
from jax import lax
from jax.experimental import pallas as pl
from jax.experimental.pallas import tpu as pltpu


def kernel(
    x,
):
    def body(*refs):
        pass

    out_shape = jax.ShapeDtypeStruct(..., jnp.float32)
    return pl.pallas_call(body, out_shape=out_shape)(...)


# baseline (device time: 18316 ns/iter reference)
import jax
import jax.numpy as jnp
from jax import lax
from jax.experimental import pallas as pl
from jax.experimental.pallas import tpu as pltpu

M = 1024
N = 1024
HALF = N // 2


def kernel(x):
    x2 = x.reshape(M, N)

    def body(x_ref, out_ref, send_buf, recv_buf, send_sem, recv_sem):
        my_x = lax.axis_index("x")
        my_y = lax.axis_index("y")
        peer = (1 - my_x, my_y)

        barrier = pltpu.get_barrier_semaphore()
        pl.semaphore_signal(
            barrier, inc=1, device_id=peer, device_id_type=pl.DeviceIdType.MESH
        )
        pl.semaphore_wait(barrier, 1)

        @pl.when(my_x == 0)
        def _():
            send_buf[:, :] = x_ref[:, HALF:].astype(jnp.bfloat16)

        @pl.when(my_x == 1)
        def _():
            send_buf[:, :] = x_ref[:, :HALF].astype(jnp.bfloat16)

        rdma = pltpu.make_async_remote_copy(
            src_ref=send_buf,
            dst_ref=recv_buf,
            send_sem=send_sem,
            recv_sem=recv_sem,
            device_id=peer,
            device_id_type=pl.DeviceIdType.MESH,
        )
        rdma.start()
        rdma.wait()

        @pl.when(my_x == 0)
        def _():
            out_ref[:, :] = x_ref[:, :HALF].astype(jnp.bfloat16) + recv_buf[:, :]

        @pl.when(my_x == 1)
        def _():
            out_ref[:, :] = x_ref[:, HALF:].astype(jnp.bfloat16) + recv_buf[:, :]

    return pl.pallas_call(
        body,
        out_shape=jax.ShapeDtypeStruct((M, HALF), jnp.bfloat16),
        in_specs=[pl.BlockSpec(memory_space=pltpu.VMEM)],
        out_specs=pl.BlockSpec(memory_space=pltpu.VMEM),
        scratch_shapes=[
            pltpu.VMEM((M, HALF), jnp.bfloat16),
            pltpu.VMEM((M, HALF), jnp.bfloat16),
            pltpu.SemaphoreType.DMA,
            pltpu.SemaphoreType.DMA,
        ],
        compiler_params=pltpu.CompilerParams(collective_id=0),
    )(x2)


# device time: 16534 ns/iter; 1.1078x vs baseline; 1.1078x over previous
import jax
import jax.numpy as jnp
from jax import lax
from jax.experimental import pallas as pl
from jax.experimental.pallas import tpu as pltpu

M = 1024
N = 1024
HALF = N // 2
Q = N // 4
K = 4
CM = M // K


def kernel(x):
    x2 = x.reshape(M, N)

    def body(
        x_ref,
        out_ref,
        send_x,
        recv_x,
        loc,
        fbuf,
        gbuf,
        x_send_sems,
        x_recv_sems,
        y_send_sems,
        y_recv_sems,
    ):
        my_x = lax.axis_index("x")
        my_y = lax.axis_index("y")
        x_peer = (1 - my_x, my_y)
        y_peer = (my_x, 1 - my_y)

        barrier = pltpu.get_barrier_semaphore()
        for peer in (x_peer, y_peer):
            pl.semaphore_signal(
                barrier,
                inc=1,
                device_id=peer,
                device_id_type=pl.DeviceIdType.MESH,
            )
        pl.semaphore_wait(barrier, 2)

        def x_rdma(k):
            rows = pl.ds(k * CM, CM)
            return pltpu.make_async_remote_copy(
                src_ref=send_x.at[rows],
                dst_ref=recv_x.at[rows],
                send_sem=x_send_sems.at[k],
                recv_sem=x_recv_sems.at[k],
                device_id=x_peer,
                device_id_type=pl.DeviceIdType.MESH,
            )

        def y_rdma(k):
            rows = pl.ds(k * CM, CM)
            return pltpu.make_async_remote_copy(
                src_ref=fbuf.at[rows],
                dst_ref=gbuf.at[rows],
                send_sem=y_send_sems.at[k],
                recv_sem=y_recv_sems.at[k],
                device_id=y_peer,
                device_id_type=pl.DeviceIdType.MESH,
            )

        for k in range(K):
            rows = pl.ds(k * CM, CM)
            for px, py in ((0, 0), (0, 1), (1, 0), (1, 1)):
                c_me = px * HALF + py * Q
                c_xp = (1 - px) * HALF + py * Q

                @pl.when(jnp.logical_and(my_x == px, my_y == py))
                def _(rows=rows, c_me=c_me, c_xp=c_xp):
                    send_x[rows, :] = x_ref[rows, c_xp : c_xp + Q].astype(
                        jnp.bfloat16
                    )
                    loc[rows, :] = x_ref[rows, c_me : c_me + Q].astype(
                        jnp.bfloat16
                    )

            x_rdma(k).start()

        for k in range(K):
            rows = pl.ds(k * CM, CM)
            x_rdma(k).wait_recv()
            fbuf[rows, :] = loc[rows, :] + recv_x[rows, :]
            y_rdma(k).start()

        for k in range(K):
            y_rdma(k).wait_recv()

        @pl.when(my_y == 0)
        def _():
            out_ref[:, :Q] = fbuf[:, :]
            out_ref[:, Q:] = gbuf[:, :]

        @pl.when(my_y == 1)
        def _():
            out_ref[:, :Q] = gbuf[:, :]
            out_ref[:, Q:] = fbuf[:, :]

        for k in range(K):
            x_rdma(k).wait_send()
            y_rdma(k).wait_send()

    return pl.pallas_call(
        body,
        out_shape=jax.ShapeDtypeStruct((M, HALF), jnp.bfloat16),
        in_specs=[pl.BlockSpec(memory_space=pltpu.VMEM)],
        out_specs=pl.BlockSpec(memory_space=pltpu.VMEM),
        scratch_shapes=[
            pltpu.VMEM((M, Q), jnp.bfloat16),
            pltpu.VMEM((M, Q), jnp.bfloat16),
            pltpu.VMEM((M, Q), jnp.bfloat16),
            pltpu.VMEM((M, Q), jnp.bfloat16),
            pltpu.VMEM((M, Q), jnp.bfloat16),
            pltpu.SemaphoreType.DMA((K,)),
            pltpu.SemaphoreType.DMA((K,)),
            pltpu.SemaphoreType.DMA((K,)),
            pltpu.SemaphoreType.DMA((K,)),
        ],
        compiler_params=pltpu.CompilerParams(collective_id=0),
    )(x2)


# device time: 16128 ns/iter; 1.1357x vs baseline; 1.0252x over previous
import jax
import jax.numpy as jnp
from jax import lax
from jax.experimental import pallas as pl
from jax.experimental.pallas import tpu as pltpu

M = 1024
N = 1024
HALF = N // 2
Q = N // 4
K = 4
CM = M // K


def kernel(x):
    x2 = x.reshape(M, N)

    def body(x_ref, out_ref, send_x, recv_x, x_send_sems, x_recv_sems,
             y_send_sems, y_recv_sems):
        my_x = lax.axis_index("x")
        my_y = lax.axis_index("y")
        x_peer = (1 - my_x, my_y)
        y_peer = (my_x, 1 - my_y)

        barrier = pltpu.get_barrier_semaphore()
        for peer in (x_peer, y_peer):
            pl.semaphore_signal(
                barrier,
                inc=1,
                device_id=peer,
                device_id_type=pl.DeviceIdType.MESH,
            )
        pl.semaphore_wait(barrier, 2)

        def x_rdma(k):
            rows = pl.ds(k * CM, CM)
            return pltpu.make_async_remote_copy(
                src_ref=send_x.at[rows],
                dst_ref=recv_x.at[rows],
                send_sem=x_send_sems.at[k],
                recv_sem=x_recv_sems.at[k],
                device_id=x_peer,
                device_id_type=pl.DeviceIdType.MESH,
            )

        def y_rdma(k):
            rows = pl.ds(k * CM, CM)
            cols = pl.ds(my_y * Q, Q)
            return pltpu.make_async_remote_copy(
                src_ref=out_ref.at[rows, cols],
                dst_ref=out_ref.at[rows, cols],
                send_sem=y_send_sems.at[k],
                recv_sem=y_recv_sems.at[k],
                device_id=y_peer,
                device_id_type=pl.DeviceIdType.MESH,
            )

        for k in range(K):
            rows = pl.ds(k * CM, CM)
            for px, py in ((0, 0), (0, 1), (1, 0), (1, 1)):
                c_xp = (1 - px) * HALF + py * Q

                @pl.when(jnp.logical_and(my_x == px, my_y == py))
                def _(rows=rows, c_xp=c_xp):
                    send_x[rows, :] = x_ref[rows, c_xp : c_xp + Q].astype(
                        jnp.bfloat16
                    )

            x_rdma(k).start()

        for k in range(K):
            rows = pl.ds(k * CM, CM)
            x_rdma(k).wait_recv()
            for px, py in ((0, 0), (0, 1), (1, 0), (1, 1)):
                c_me = px * HALF + py * Q
                off = py * Q

                @pl.when(jnp.logical_and(my_x == px, my_y == py))
                def _(rows=rows, c_me=c_me, off=off):
                    out_ref[rows, off : off + Q] = (
                        x_ref[rows, c_me : c_me + Q].astype(jnp.bfloat16)
                        + recv_x[rows, :]
                    )

            y_rdma(k).start()

        for k in range(K):
            y_rdma(k).wait_recv()

        for k in range(K):
            x_rdma(k).wait_send()
            y_rdma(k).wait_send()

    return pl.pallas_call(
        body,
        out_shape=jax.ShapeDtypeStruct((M, HALF), jnp.bfloat16),
        in_specs=[pl.BlockSpec(memory_space=pltpu.VMEM)],
        out_specs=pl.BlockSpec(memory_space=pltpu.VMEM),
        scratch_shapes=[
            pltpu.VMEM((M, Q), jnp.bfloat16),
            pltpu.VMEM((M, Q), jnp.bfloat16),
            pltpu.SemaphoreType.DMA((K,)),
            pltpu.SemaphoreType.DMA((K,)),
            pltpu.SemaphoreType.DMA((K,)),
            pltpu.SemaphoreType.DMA((K,)),
        ],
        compiler_params=pltpu.CompilerParams(collective_id=0),
    )(x2)


# device time: 15837 ns/iter; 1.1565x vs baseline; 1.0184x over previous
import jax
import jax.numpy as jnp
from jax import lax
from jax.experimental import pallas as pl
from jax.experimental.pallas import tpu as pltpu

M = 1024
N = 1024
HALF = N // 2
Q = N // 4
K = 8
CM = M // K

_COORDS = ((0, 0), (0, 1), (1, 0), (1, 1))


def kernel(x):
    def body(x_ref, out_ref, send_x, recv_x, x_send_sems, x_recv_sems,
             y_send_sems, y_recv_sems):
        my_x = lax.axis_index("x")
        my_y = lax.axis_index("y")
        x_peer = (1 - my_x, my_y)
        y_peer = (my_x, 1 - my_y)

        def x_rdma(k):
            rows = pl.ds(k * CM, CM)
            return pltpu.make_async_remote_copy(
                src_ref=send_x.at[rows],
                dst_ref=recv_x.at[rows],
                send_sem=x_send_sems.at[k],
                recv_sem=x_recv_sems.at[k],
                device_id=x_peer,
                device_id_type=pl.DeviceIdType.MESH,
            )

        def y_rdma(k):
            rows = pl.ds(k * CM, CM)
            cols = pl.ds(my_y * Q, Q)
            return pltpu.make_async_remote_copy(
                src_ref=out_ref.at[rows, cols],
                dst_ref=out_ref.at[rows, cols],
                send_sem=y_send_sems.at[k],
                recv_sem=y_recv_sems.at[k],
                device_id=y_peer,
                device_id_type=pl.DeviceIdType.MESH,
            )

        def stage(k):
            rows = pl.ds(k * CM, CM)
            for px, py in _COORDS:
                c_xp = (1 - px) * HALF + py * Q

                @pl.when(jnp.logical_and(my_x == px, my_y == py))
                def _(rows=rows, c_xp=c_xp):
                    send_x[rows, :] = x_ref[0, rows, c_xp : c_xp + Q].astype(
                        jnp.bfloat16
                    )

        barrier = pltpu.get_barrier_semaphore()
        for peer in (x_peer, y_peer):
            pl.semaphore_signal(
                barrier,
                inc=1,
                device_id=peer,
                device_id_type=pl.DeviceIdType.MESH,
            )
        stage(0)
        pl.semaphore_wait(barrier, 2)

        for k in range(K):
            x_rdma(k).start()
            if k + 1 < K:
                stage(k + 1)

        for k in range(K):
            rows = pl.ds(k * CM, CM)
            x_rdma(k).wait_recv()
            for px, py in _COORDS:
                c_me = px * HALF + py * Q
                off = py * Q

                @pl.when(jnp.logical_and(my_x == px, my_y == py))
                def _(rows=rows, c_me=c_me, off=off):
                    out_ref[rows, off : off + Q] = (
                        x_ref[0, rows, c_me : c_me + Q].astype(jnp.bfloat16)
                        + recv_x[rows, :]
                    )

            y_rdma(k).start()

        for k in range(K):
            y_rdma(k).wait_recv()

        for k in range(K):
            x_rdma(k).wait_send()
            y_rdma(k).wait_send()

    return pl.pallas_call(
        body,
        out_shape=jax.ShapeDtypeStruct((M, HALF), jnp.bfloat16),
        in_specs=[pl.BlockSpec(memory_space=pltpu.VMEM)],
        out_specs=pl.BlockSpec(memory_space=pltpu.VMEM),
        scratch_shapes=[
            pltpu.VMEM((M, Q), jnp.bfloat16),
            pltpu.VMEM((M, Q), jnp.bfloat16),
            pltpu.SemaphoreType.DMA((K,)),
            pltpu.SemaphoreType.DMA((K,)),
            pltpu.SemaphoreType.DMA((K,)),
            pltpu.SemaphoreType.DMA((K,)),
        ],
        compiler_params=pltpu.CompilerParams(collective_id=0),
    )(x)
